# baseline (speedup 1.0000x reference)
import functools

import jax
import jax.numpy as jnp
from jax import lax
from jax.experimental import pallas as pl
from jax.experimental.pallas import tpu as pltpu
from jax.experimental.pallas import tpu_sc as plsc


def _gather_kernel(n_total, d, chunk, n_workers, nc):
    n_per_w = n_total // n_workers
    n_chunks = n_per_w // chunk
    mesh = plsc.VectorSubcoreMesh(core_axis_name="c", subcore_axis_name="s")

    @functools.partial(
        pl.kernel,
        mesh=mesh,
        compiler_params=pltpu.CompilerParams(use_tc_tiling_on_sc=False),
        out_type=jax.ShapeDtypeStruct((n_total, d), jnp.float32),
        scratch_types=[
            pltpu.VMEM((chunk,), jnp.int32),
            pltpu.VMEM((chunk, d), jnp.float32),
            pltpu.SemaphoreType.DMA,
        ],
    )
    def k(table_hbm, idx_hbm, out_hbm, idx_v, rows_v, sem):
        wid = lax.axis_index("s") * nc + lax.axis_index("c")
        base = wid * n_per_w

        def body(i, carry):
            off = base + i * chunk
            pltpu.sync_copy(idx_hbm.at[pl.ds(off, chunk)], idx_v)
            pltpu.async_copy(table_hbm.at[idx_v], rows_v, sem).wait()
            pltpu.sync_copy(rows_v, out_hbm.at[pl.ds(off, chunk)])
            return carry

        lax.fori_loop(0, n_chunks, body, 0)

    return k


def kernel(x, table):
    b, s = x.shape
    v, d = table.shape
    n_total = b * s

    info = plsc.get_sparse_core_info()
    nc, ns = info.num_cores, info.num_subcores
    n_workers = nc * ns

    chunk = 512
    flat_idx = x.reshape(n_total).astype(jnp.int32)
    out = _gather_kernel(n_total, d, chunk, n_workers, nc)(table, flat_idx)
    return out.reshape(b, s, d)


# speedup vs baseline: 1.0478x; 1.0478x over previous
import functools

import jax
import jax.numpy as jnp
from jax import lax
from jax.experimental import pallas as pl
from jax.experimental.pallas import tpu as pltpu
from jax.experimental.pallas import tpu_sc as plsc


def _gather_kernel(n_total, d, chunk, n_workers, nc):
    n_per_w = n_total // n_workers
    n_chunks = n_per_w // chunk
    assert n_chunks % 2 == 0
    n_outer = n_chunks // 2
    mesh = plsc.VectorSubcoreMesh(core_axis_name="c", subcore_axis_name="s")

    @functools.partial(
        pl.kernel,
        mesh=mesh,
        compiler_params=pltpu.CompilerParams(use_tc_tiling_on_sc=False),
        out_type=jax.ShapeDtypeStruct((n_total, d), jnp.float32),
        scratch_types=[
            pltpu.VMEM((n_per_w,), jnp.int32),
            pltpu.VMEM((chunk, d), jnp.float32),
            pltpu.VMEM((chunk, d), jnp.float32),
            pltpu.SemaphoreType.DMA,
            pltpu.SemaphoreType.DMA,
            pltpu.SemaphoreType.DMA,
            pltpu.SemaphoreType.DMA,
        ],
    )
    def k(table_hbm, idx_hbm, out_hbm, idx_v, rows0, rows1, g0, g1, s0, s1):
        wid = lax.axis_index("s") * nc + lax.axis_index("c")
        base = wid * n_per_w

        pltpu.sync_copy(idx_hbm.at[pl.ds(base, n_per_w)], idx_v)

        def gather(c, rows, sem):
            pltpu.make_async_copy(
                table_hbm.at[idx_v.at[pl.ds(c * chunk, chunk)]], rows, sem
            ).start()

        def store(c, rows, sem):
            pltpu.make_async_copy(
                rows, out_hbm.at[pl.ds(base + c * chunk, chunk)], sem
            ).start()

        def wait_g(rows, sem):
            pltpu.make_async_copy(table_hbm.at[idx_v.at[pl.ds(0, chunk)]], rows, sem).wait()

        def wait_s(rows, sem):
            pltpu.make_async_copy(rows, out_hbm.at[pl.ds(base, chunk)], sem).wait()

        gather(0, rows0, g0)

        def body(j, carry):
            @pl.when(j > 0)
            def _():
                wait_s(rows1, s1)

            gather(2 * j + 1, rows1, g1)

            wait_g(rows0, g0)
            store(2 * j, rows0, s0)

            @pl.when(j < n_outer - 1)
            def _():
                wait_s(rows0, s0)
                gather(2 * j + 2, rows0, g0)

            wait_g(rows1, g1)
            store(2 * j + 1, rows1, s1)
            return carry

        lax.fori_loop(0, n_outer, body, 0)
        wait_s(rows0, s0)
        wait_s(rows1, s1)

    return k


def kernel(x, table):
    b, s = x.shape
    v, d = table.shape
    n_total = b * s

    info = plsc.get_sparse_core_info()
    nc, ns = info.num_cores, info.num_subcores
    n_workers = nc * ns

    chunk = 512
    flat_idx = x.reshape(n_total).astype(jnp.int32)
    out = _gather_kernel(n_total, d, chunk, n_workers, nc)(table, flat_idx)
    return out.reshape(b, s, d)
